# Initial kernel scaffold; baseline (speedup 1.0000x reference)
import functools

import jax
import jax.numpy as jnp
from jax import lax
from jax.experimental import pallas as pl
from jax.experimental.pallas import tpu as pltpu
from jax.experimental.pallas import tpu_sc as plsc

NUM_WORKERS = 32
CHUNK = 128


def _embedding_lookup(idx3, table):
    nw, n_chunks, chunk = idx3.shape
    b_per_w = n_chunks * chunk
    total = nw * b_per_w
    d = table.shape[1]
    mesh = plsc.VectorSubcoreMesh(core_axis_name="c", subcore_axis_name="s")

    @functools.partial(
        pl.kernel,
        out_type=jax.ShapeDtypeStruct((total, d), jnp.float32),
        mesh=mesh,
        scratch_types=[
            pltpu.VMEM((n_chunks, chunk), jnp.int32),
            pltpu.VMEM((2, chunk, d), jnp.float32),
            pltpu.SemaphoreType.DMA,
        ],
    )
    def body(idx_hbm, table_hbm, out_hbm, idx_v, rows_v, gsem):
        wid = lax.axis_index("s") * 2 + lax.axis_index("c")
        base = wid * b_per_w
        pltpu.sync_copy(idx_hbm.at[wid], idx_v)
        pltpu.async_copy(table_hbm.at[idx_v.at[0]], rows_v.at[0], gsem)

        def step(j, carry):
            slot = lax.rem(j, 2)

            @pl.when(j + 1 < n_chunks)
            def _():
                pltpu.async_copy(
                    table_hbm.at[idx_v.at[j + 1]], rows_v.at[1 - slot], gsem
                )

            pltpu.make_async_copy(
                table_hbm.at[idx_v.at[j]], rows_v.at[slot], gsem
            ).wait()
            pltpu.sync_copy(
                rows_v.at[slot], out_hbm.at[pl.ds(base + j * chunk, chunk)]
            )
            return carry

        lax.fori_loop(0, n_chunks, step, 0)

    return body(idx3, table)


def kernel(tokens, table):
    s, l = tokens.shape
    d = table.shape[1]
    total = s * l
    b_per_w = total // NUM_WORKERS
    n_chunks = b_per_w // CHUNK
    idx3 = tokens.astype(jnp.int32).reshape(NUM_WORKERS, n_chunks, CHUNK)
    out = _embedding_lookup(idx3, table)
    return out.reshape(s, l, d)


# speedup vs baseline: 1.4250x; 1.4250x over previous
import functools

import jax
import jax.numpy as jnp
from jax import lax
from jax.experimental import pallas as pl
from jax.experimental.pallas import tpu as pltpu
from jax.experimental.pallas import tpu_sc as plsc

NUM_WORKERS = 32
CHUNK = 128


def _embedding_lookup(idx3, table):
    nw, n_chunks, chunk = idx3.shape
    b_per_w = n_chunks * chunk
    total = nw * b_per_w
    d = table.shape[1]
    mesh = plsc.VectorSubcoreMesh(core_axis_name="c", subcore_axis_name="s")

    @functools.partial(
        pl.kernel,
        out_type=jax.ShapeDtypeStruct((total, d), jnp.float32),
        mesh=mesh,
        scratch_types=[
            pltpu.VMEM((n_chunks, chunk), jnp.int32),
            pltpu.VMEM((2, chunk, d), jnp.float32),
            pltpu.SemaphoreType.DMA,
        ],
        compiler_params=pltpu.CompilerParams(use_tc_tiling_on_sc=False),
    )
    def body(idx_hbm, table_hbm, out_hbm, idx_v, rows_v, gsem):
        wid = lax.axis_index("s") * 2 + lax.axis_index("c")
        base = wid * b_per_w
        pltpu.sync_copy(idx_hbm.at[wid], idx_v)
        pltpu.async_copy(table_hbm.at[idx_v.at[0]], rows_v.at[0], gsem)

        def step(j, carry):
            slot = lax.rem(j, 2)

            @pl.when(j + 1 < n_chunks)
            def _():
                pltpu.async_copy(
                    table_hbm.at[idx_v.at[j + 1]], rows_v.at[1 - slot], gsem
                )

            pltpu.make_async_copy(
                table_hbm.at[idx_v.at[j]], rows_v.at[slot], gsem
            ).wait()
            pltpu.sync_copy(
                rows_v.at[slot], out_hbm.at[pl.ds(base + j * chunk, chunk)]
            )
            return carry

        lax.fori_loop(0, n_chunks, step, 0)

    return body(idx3, table)


def kernel(tokens, table):
    s, l = tokens.shape
    d = table.shape[1]
    total = s * l
    b_per_w = total // NUM_WORKERS
    n_chunks = b_per_w // CHUNK
    idx3 = tokens.astype(jnp.int32).reshape(NUM_WORKERS, n_chunks, CHUNK)
    out = _embedding_lookup(idx3, table)
    return out.reshape(s, l, d)


# speedup vs baseline: 1.5027x; 1.0545x over previous
import functools

import jax
import jax.numpy as jnp
from jax import lax
from jax.experimental import pallas as pl
from jax.experimental.pallas import tpu as pltpu
from jax.experimental.pallas import tpu_sc as plsc

NUM_WORKERS = 32
CHUNK = 1024


def _embedding_lookup(idx3, table):
    nw, n_chunks, chunk = idx3.shape
    b_per_w = n_chunks * chunk
    total = nw * b_per_w
    d = table.shape[1]
    mesh = plsc.VectorSubcoreMesh(core_axis_name="c", subcore_axis_name="s")

    @functools.partial(
        pl.kernel,
        out_type=jax.ShapeDtypeStruct((total, d), jnp.float32),
        mesh=mesh,
        scratch_types=[
            pltpu.VMEM((n_chunks, chunk), jnp.int32),
            pltpu.VMEM((2, chunk, d), jnp.float32),
            pltpu.SemaphoreType.DMA,
        ],
        compiler_params=pltpu.CompilerParams(use_tc_tiling_on_sc=False),
    )
    def body(idx_hbm, table_hbm, out_hbm, idx_v, rows_v, gsem):
        wid = lax.axis_index("s") * 2 + lax.axis_index("c")
        base = wid * b_per_w
        pltpu.sync_copy(idx_hbm.at[wid], idx_v)
        pltpu.async_copy(table_hbm.at[idx_v.at[0]], rows_v.at[0], gsem)

        def step(j, carry):
            slot = lax.rem(j, 2)

            @pl.when(j + 1 < n_chunks)
            def _():
                pltpu.async_copy(
                    table_hbm.at[idx_v.at[j + 1]], rows_v.at[1 - slot], gsem
                )

            pltpu.make_async_copy(
                table_hbm.at[idx_v.at[j]], rows_v.at[slot], gsem
            ).wait()
            pltpu.sync_copy(
                rows_v.at[slot], out_hbm.at[pl.ds(base + j * chunk, chunk)]
            )
            return carry

        lax.fori_loop(0, n_chunks, step, 0)

    return body(idx3, table)


def kernel(tokens, table):
    s, l = tokens.shape
    d = table.shape[1]
    total = s * l
    b_per_w = total // NUM_WORKERS
    n_chunks = b_per_w // CHUNK
    idx3 = tokens.astype(jnp.int32).reshape(NUM_WORKERS, n_chunks, CHUNK)
    out = _embedding_lookup(idx3, table)
    return out.reshape(s, l, d)
